# baseline (speedup 1.0000x reference)
import functools

import jax
import jax.numpy as jnp
from jax import lax
from jax.experimental import pallas as pl
from jax.experimental.pallas import tpu as pltpu
from jax.experimental.pallas import tpu_sc as plsc

NUM_CORES = 2
NUM_SUBCORES = 16
NUM_WORKERS = NUM_CORES * NUM_SUBCORES

D = 64
N = 4096 * 200
PER_W = N // NUM_WORKERS
CHUNK = 1024
N_CHUNKS = PER_W // CHUNK


def _body(feat_hbm, idx_hbm, table_hbm, out_hbm, idx_v, buf_v, sem):
    c = lax.axis_index("c")
    s = lax.axis_index("s")
    wid = s * NUM_CORES + c
    base = wid * PER_W

    def step(i, carry):
        off = base + i * CHUNK
        pltpu.sync_copy(idx_hbm.at[pl.ds(off, CHUNK)], idx_v)
        pltpu.sync_copy(feat_hbm.at[pl.ds(off, CHUNK)], buf_v)
        pltpu.async_copy(table_hbm.at[idx_v], buf_v, sem, add=True).wait()
        pltpu.sync_copy(buf_v, out_hbm.at[pl.ds(off, CHUNK)])
        return carry

    lax.fori_loop(0, N_CHUNKS, step, 0)


@jax.jit
def _run(feat2d, idx1d, table):
    mesh = plsc.VectorSubcoreMesh(core_axis_name="c", subcore_axis_name="s")
    kern = pl.kernel(
        _body,
        out_type=jax.ShapeDtypeStruct((N, D), jnp.float32),
        mesh=mesh,
        scratch_types=[
            pltpu.VMEM((CHUNK,), jnp.int32),
            pltpu.VMEM((CHUNK, D), jnp.float32),
            pltpu.SemaphoreType.DMA,
        ],
        compiler_params=pltpu.CompilerParams(use_tc_tiling_on_sc=False),
    )
    return kern(feat2d, idx1d, table)


def kernel(features, indices, table):
    B, H, d = features.shape
    feat2d = features.reshape(B * H, d)
    idx1d = indices.reshape(-1).astype(jnp.int32)
    out = _run(feat2d, idx1d, table)
    return out.reshape(B, H, d)


# speedup vs baseline: 1.0221x; 1.0221x over previous
import functools

import jax
import jax.numpy as jnp
from jax import lax
from jax.experimental import pallas as pl
from jax.experimental.pallas import tpu as pltpu
from jax.experimental.pallas import tpu_sc as plsc

NUM_CORES = 2
NUM_SUBCORES = 16
NUM_WORKERS = NUM_CORES * NUM_SUBCORES

D = 64
N = 4096 * 200
PER_W = N // NUM_WORKERS
CHUNK = 400
NCH = PER_W // CHUNK
NBUF = 4
NGRP = NCH // NBUF


def _body(feat_hbm, idx_hbm, table_hbm, out_hbm, idx_v, buf_v, *sems):
    lsem = sems[0:NBUF]
    gsem = sems[NBUF:2 * NBUF]
    ssem = sems[2 * NBUF:3 * NBUF]

    c = lax.axis_index("c")
    s = lax.axis_index("s")
    wid = s * NUM_CORES + c
    base = wid * PER_W

    def chunk_off(i):
        return base + i * CHUNK

    def load_issue(i, b):
        pltpu.async_copy(feat_hbm.at[pl.ds(chunk_off(i), CHUNK)],
                         buf_v.at[b], lsem[b])

    def load_wait(i, b):
        pltpu.make_async_copy(feat_hbm.at[pl.ds(chunk_off(i), CHUNK)],
                              buf_v.at[b], lsem[b]).wait()

    def gather_issue(i, b):
        pltpu.async_copy(table_hbm.at[idx_v.at[pl.ds(i * CHUNK, CHUNK)]],
                         buf_v.at[b], gsem[b], add=True)

    def gather_wait(i, b):
        pltpu.make_async_copy(table_hbm.at[idx_v.at[pl.ds(i * CHUNK, CHUNK)]],
                              buf_v.at[b], gsem[b]).wait()

    def store_issue(i, b):
        pltpu.async_copy(buf_v.at[b], out_hbm.at[pl.ds(chunk_off(i), CHUNK)],
                         ssem[b])

    def store_wait(i, b):
        pltpu.make_async_copy(buf_v.at[b], out_hbm.at[pl.ds(chunk_off(i), CHUNK)],
                              ssem[b]).wait()

    pltpu.sync_copy(idx_hbm.at[pl.ds(base, PER_W)], idx_v)

    load_issue(0, 0)
    load_issue(1, 1)

    def group(g, carry):
        for b in range(NBUF):
            i = g * NBUF + b
            bp2 = (b + 2) % NBUF
            bm1 = (b - 1) % NBUF

            @pl.when(i + 2 < NCH)
            def _():
                @pl.when(i >= 2)
                def _():
                    store_wait(i - 2, bp2)
                load_issue(i + 2, bp2)

            load_wait(i, b)
            gather_issue(i, b)

            @pl.when(i >= 1)
            def _():
                gather_wait(i - 1, bm1)
                store_issue(i - 1, bm1)
        return carry

    lax.fori_loop(0, NGRP, group, 0)

    last = NCH - 1
    gather_wait(last, last % NBUF)
    store_issue(last, last % NBUF)
    for j in range(NCH - 4, NCH):
        store_wait(j, j % NBUF)


@jax.jit
def _run(feat2d, idx1d, table):
    mesh = plsc.VectorSubcoreMesh(core_axis_name="c", subcore_axis_name="s")
    kern = pl.kernel(
        _body,
        out_type=jax.ShapeDtypeStruct((N, D), jnp.float32),
        mesh=mesh,
        scratch_types=[
            pltpu.VMEM((PER_W,), jnp.int32),
            pltpu.VMEM((NBUF, CHUNK, D), jnp.float32),
        ] + [pltpu.SemaphoreType.DMA] * (3 * NBUF),
        compiler_params=pltpu.CompilerParams(use_tc_tiling_on_sc=False),
    )
    return kern(feat2d, idx1d, table)


def kernel(features, indices, table):
    B, H, d = features.shape
    feat2d = features.reshape(B * H, d)
    idx1d = indices.reshape(-1).astype(jnp.int32)
    out = _run(feat2d, idx1d, table)
    return out.reshape(B, H, d)
